# baseline (speedup 1.0000x reference)
import functools

import jax
import jax.numpy as jnp
from jax import lax
from jax.experimental import pallas as pl
from jax.experimental.pallas import tpu as pltpu
from jax.experimental.pallas import tpu_sc as plsc

BATCH = 4096
MAXLEN = 200
EMBED_DIM = 64

NUM_CORES = 2
NUM_SUBCORES = 16
NUM_WORKERS = NUM_CORES * NUM_SUBCORES

TOTAL_ROWS = BATCH * MAXLEN
ROWS_PER_WORKER = TOTAL_ROWS // NUM_WORKERS
CHUNK = 128
CHUNKS_PER_WORKER = ROWS_PER_WORKER // CHUNK

POS_WORDS = MAXLEN * EMBED_DIM
CHUNK_WORDS = CHUNK * EMBED_DIM
POS_EXT_WORDS = POS_WORDS + CHUNK_WORDS - 512


def _sc_body(idx_hbm, tok_hbm, pos_hbm, out_hbm, idx_v, rows_v, pos_v, sem):
    wid = lax.axis_index("s") * NUM_CORES + lax.axis_index("c")
    worker_base = wid * ROWS_PER_WORKER

    pltpu.sync_copy(pos_hbm, pos_v.at[pl.ds(0, POS_WORDS)])
    pltpu.sync_copy(pos_hbm.at[pl.ds(0, POS_EXT_WORDS - POS_WORDS)],
                    pos_v.at[pl.ds(POS_WORDS, POS_EXT_WORDS - POS_WORDS)])

    def chunk_body(j, carry):
        base = worker_base + j * CHUNK
        pltpu.sync_copy(idx_hbm.at[pl.ds(base, CHUNK)], idx_v)
        pltpu.async_copy(tok_hbm.at[idx_v], rows_v, sem).wait()

        p0 = lax.rem(base, MAXLEN) * EMBED_DIM

        def add_body(r, c2):
            pr = p0 + r * EMBED_DIM
            for c in range(EMBED_DIM // 16):
                v = pos_v[pl.ds(pr + c * 16, 16)]
                plsc.addupdate(rows_v.at[r, pl.ds(c * 16, 16)], v)
            return c2

        lax.fori_loop(0, CHUNK, add_body, 0, unroll=4)
        pltpu.sync_copy(rows_v, out_hbm.at[pl.ds(base, CHUNK)])
        return carry

    lax.fori_loop(0, CHUNKS_PER_WORKER, chunk_body, 0)


@jax.jit
def kernel(x, token_table, pos_table):
    idx = x.reshape(-1).astype(jnp.int32)
    pos_flat = pos_table.reshape(-1)

    mesh = plsc.VectorSubcoreMesh(
        core_axis_name="c", subcore_axis_name="s",
        num_cores=NUM_CORES, num_subcores=NUM_SUBCORES)
    run = pl.kernel(
        _sc_body,
        out_type=jax.ShapeDtypeStruct((TOTAL_ROWS, EMBED_DIM), jnp.float32),
        mesh=mesh,
        scratch_types=[
            pltpu.VMEM((CHUNK,), jnp.int32),
            pltpu.VMEM((CHUNK, EMBED_DIM), jnp.float32),
            pltpu.VMEM((POS_EXT_WORDS,), jnp.float32),
            pltpu.SemaphoreType.DMA,
        ],
        compiler_params=pltpu.CompilerParams(use_tc_tiling_on_sc=False),
    )
    out = run(idx, token_table, pos_flat)
    return out.reshape(BATCH, MAXLEN, EMBED_DIM)


# speedup vs baseline: 1.1683x; 1.1683x over previous
import jax
import jax.numpy as jnp
from jax import lax
from jax.experimental import pallas as pl
from jax.experimental.pallas import tpu as pltpu
from jax.experimental.pallas import tpu_sc as plsc

BATCH = 4096
MAXLEN = 200
EMBED_DIM = 64

NUM_CORES = 2
NUM_SUBCORES = 16
NUM_WORKERS = NUM_CORES * NUM_SUBCORES

TOTAL_ROWS = BATCH * MAXLEN
ROWS_PER_WORKER = TOTAL_ROWS // NUM_WORKERS
STREAM = 128
NSTREAM = 4
CHUNK = STREAM * NSTREAM
NCHUNK = ROWS_PER_WORKER // CHUNK
POS_WORDS = MAXLEN * EMBED_DIM


def _sc_body(idx_hbm, tok_hbm, pos_hbm, out_hbm,
             idx_v, rows_v, pos_v, sem_g0, sem_g1, sem_s0, sem_s1):
    sem_g = (sem_g0, sem_g1)
    sem_s = (sem_s0, sem_s1)
    wid = lax.axis_index("s") * NUM_CORES + lax.axis_index("c")
    worker_base = wid * ROWS_PER_WORKER

    pltpu.sync_copy(pos_hbm, pos_v)

    def start_gather(j, b):
        base = worker_base + j * CHUNK
        for s in range(NSTREAM):
            pltpu.sync_copy(idx_hbm.at[pl.ds(base + s * STREAM, STREAM)],
                            idx_v.at[b, s])
        for s in range(NSTREAM):
            pltpu.async_copy(tok_hbm.at[idx_v.at[b, s]],
                             rows_v.at[b, pl.ds(s * STREAM, STREAM)],
                             sem_g[b])

    def wait_gather(b):
        pltpu.make_async_copy(tok_hbm.at[pl.ds(0, CHUNK)],
                              rows_v.at[b], sem_g[b]).wait()

    def start_scatter(j, b):
        base = worker_base + j * CHUNK
        pltpu.async_copy(rows_v.at[b], out_hbm.at[pl.ds(base, CHUNK)],
                         sem_s[b])

    def wait_scatter(b):
        pltpu.make_async_copy(rows_v.at[b], out_hbm.at[pl.ds(0, CHUNK)],
                              sem_s[b]).wait()

    def add_pos(j, b):
        base = worker_base + j * CHUNK
        p0 = lax.rem(base, MAXLEN) * EMBED_DIM

        def add_body(r, pr):
            for c in range(EMBED_DIM // 16):
                v = pos_v[pl.ds(pr + c * 16, 16)]
                plsc.addupdate(rows_v.at[b, r, pl.ds(c * 16, 16)], v)
            prn = pr + EMBED_DIM
            return lax.select(prn >= POS_WORDS, prn - POS_WORDS, prn)

        lax.fori_loop(0, CHUNK, add_body, p0, unroll=8)

    start_gather(0, 0)

    def outer_body(j2, carry):
        for b in (0, 1):
            j = j2 * 2 + b
            @pl.when(j >= 1)
            def _():
                wait_scatter(1 - b)

            @pl.when(j + 1 < NCHUNK)
            def _():
                start_gather(j + 1, 1 - b)

            wait_gather(b)
            add_pos(j, b)
            start_scatter(j, b)
        return carry

    lax.fori_loop(0, NCHUNK // 2, outer_body, 0)
    wait_scatter(1)


@jax.jit
def kernel(x, token_table, pos_table):
    idx = x.reshape(-1).astype(jnp.int32)
    pos_flat = pos_table.reshape(-1)

    mesh = plsc.VectorSubcoreMesh(
        core_axis_name="c", subcore_axis_name="s",
        num_cores=NUM_CORES, num_subcores=NUM_SUBCORES)
    run = pl.kernel(
        _sc_body,
        out_type=jax.ShapeDtypeStruct((TOTAL_ROWS, EMBED_DIM), jnp.float32),
        mesh=mesh,
        scratch_types=[
            pltpu.VMEM((2, NSTREAM, STREAM), jnp.int32),
            pltpu.VMEM((2, CHUNK, EMBED_DIM), jnp.float32),
            pltpu.VMEM((POS_WORDS,), jnp.float32),
            pltpu.SemaphoreType.DMA,
            pltpu.SemaphoreType.DMA,
            pltpu.SemaphoreType.DMA,
            pltpu.SemaphoreType.DMA,
        ],
        compiler_params=pltpu.CompilerParams(use_tc_tiling_on_sc=False),
    )
    out = run(idx, token_table, pos_flat)
    return out.reshape(BATCH, MAXLEN, EMBED_DIM)


# speedup vs baseline: 1.4836x; 1.2699x over previous
import jax
import jax.numpy as jnp
from jax import lax
from jax.experimental import pallas as pl
from jax.experimental.pallas import tpu as pltpu
from jax.experimental.pallas import tpu_sc as plsc

BATCH = 4096
MAXLEN = 200
EMBED_DIM = 64
PAD_DIM = 128

NUM_CORES = 2
NUM_SUBCORES = 16
NUM_WORKERS = NUM_CORES * NUM_SUBCORES

B_PER_W = BATCH // NUM_WORKERS
STREAM0 = 128
STREAM1 = MAXLEN - STREAM0


def _sc_body(x_hbm, tbl, pos_hbm, out_hbm, idx_v, rows_v, acc_v, pos_v,
             sem_g0, sem_g1, sem_o0, sem_o1):
    sem_g = (sem_g0, sem_g1)
    sem_o = (sem_o0, sem_o1)
    wid = lax.axis_index("s") * NUM_CORES + lax.axis_index("c")
    b_base = wid * B_PER_W

    pltpu.sync_copy(pos_hbm, pos_v)

    def fire_gather(j, hb):
        b = b_base + j
        pltpu.sync_copy(x_hbm.at[b], idx_v.at[hb])
        pltpu.async_copy(tbl.at[idx_v.at[hb, pl.ds(0, STREAM0)]],
                         rows_v.at[hb, pl.ds(0, STREAM0)], sem_g[hb])
        pltpu.async_copy(tbl.at[idx_v.at[hb, pl.ds(STREAM0, STREAM1)]],
                         rows_v.at[hb, pl.ds(STREAM0, STREAM1)], sem_g[hb])

    def wait_g(hb):
        pltpu.make_async_copy(tbl.at[pl.ds(0, MAXLEN)], rows_v.at[hb],
                              sem_g[hb]).wait()

    def start_out(j, hb):
        b = b_base + j
        pltpu.async_copy(acc_v.at[hb], out_hbm.at[b], sem_o[hb])

    def wait_o(hb):
        pltpu.make_async_copy(acc_v.at[hb], out_hbm.at[0], sem_o[hb]).wait()

    def accumulate(hb):
        def rbody(r, carry):
            for c in range(EMBED_DIM // 16):
                v = (rows_v[hb, r, pl.ds(c * 16, 16)]
                     + pos_v[r, pl.ds(c * 16, 16)])
                acc_v[hb, r, pl.ds(c * 16, 16)] = v
            return carry

        lax.fori_loop(0, MAXLEN, rbody, 0, unroll=8)

    for hb in range(2):
        fire_gather(hb, hb)

    def step(j2, carry):
        for hb in range(2):
            j = j2 * 2 + hb
            wait_g(hb)

            @pl.when(j >= 2)
            def _():
                wait_o(hb)

            accumulate(hb)
            start_out(j, hb)

            @pl.when(j + 2 < B_PER_W)
            def _():
                fire_gather(j + 2, hb)
        return carry

    lax.fori_loop(0, B_PER_W // 2, step, 0)
    for hb in range(2):
        wait_o(hb)


@jax.jit
def kernel(x, token_table, pos_table):
    tbl_pad = jnp.pad(token_table, ((0, 0), (0, PAD_DIM - EMBED_DIM)))

    mesh = plsc.VectorSubcoreMesh(
        core_axis_name="c", subcore_axis_name="s",
        num_cores=NUM_CORES, num_subcores=NUM_SUBCORES)
    run = pl.kernel(
        _sc_body,
        out_type=jax.ShapeDtypeStruct((BATCH, MAXLEN, EMBED_DIM),
                                      jnp.float32),
        mesh=mesh,
        scratch_types=[
            pltpu.VMEM((2, MAXLEN), jnp.int32),
            pltpu.VMEM((2, MAXLEN, PAD_DIM), jnp.float32),
            pltpu.VMEM((2, MAXLEN, EMBED_DIM), jnp.float32),
            pltpu.VMEM((MAXLEN, EMBED_DIM), jnp.float32),
            pltpu.SemaphoreType.DMA,
            pltpu.SemaphoreType.DMA,
            pltpu.SemaphoreType.DMA,
            pltpu.SemaphoreType.DMA,
        ],
        compiler_params=pltpu.CompilerParams(use_tc_tiling_on_sc=True),
    )
    return run(x.astype(jnp.int32), tbl_pad, pos_table)
